# Initial kernel scaffold; baseline (speedup 1.0000x reference)
import jax
import jax.numpy as jnp
from jax.experimental import pallas as pl


def kernel(x, edge_index, batch, W_gcn, b_gcn, W1, b1, W3, b3):
    raise NotImplementedError("write your pallas kernel here")


# speedup vs baseline: 20.1649x; 20.1649x over previous
import functools

import jax
import jax.numpy as jnp
from jax import lax
from jax.experimental import pallas as pl
from jax.experimental.pallas import tpu as pltpu
from jax.experimental.pallas import tpu_sc as plsc

N = 10000
NP = 10240
DI = 128
DO = 64
E = 320000
NC = 2
NS = 16
NW = NC * NS
EB = 128
NB = 80
EP = NW * NB * EB
RPT = NP // NS

_MESH = functools.partial(
    plsc.VectorSubcoreMesh,
    core_axis_name="c", subcore_axis_name="s", num_cores=NC, num_subcores=NS,
)
_SC_PARAMS = pltpu.CompilerParams(use_tc_tiling_on_sc=False)


@functools.partial(
    pl.kernel,
    out_type=jax.ShapeDtypeStruct((NC, NP, 8), jnp.float32),
    mesh=_MESH(),
    compiler_params=_SC_PARAMS,
    scratch_types=[
        pltpu.VMEM((NB, EB), jnp.int32),
        pltpu.VMEM((EB, 8), jnp.float32),
        pltpu.VMEM_SHARED((NP, 8), jnp.float32),
    ],
)
def _cnt_kernel(dst_hbm, ones_hbm, zeros_hbm, out_hbm, dst_v, ones_v, cnt_sh):
    c = lax.axis_index("c")
    s = lax.axis_index("s")
    wid = s * NC + c
    base = s * RPT
    pltpu.sync_copy(zeros_hbm.at[pl.ds(base, RPT)], cnt_sh.at[pl.ds(base, RPT)])
    pltpu.sync_copy(ones_hbm, ones_v)
    pltpu.sync_copy(dst_hbm.at[wid], dst_v)
    plsc.subcore_barrier()

    def body(b, carry):
        pltpu.sync_copy(ones_v, cnt_sh.at[dst_v.at[b]], add=True)
        return carry

    lax.fori_loop(0, NB, body, 0)
    plsc.subcore_barrier()
    pltpu.sync_copy(cnt_sh.at[pl.ds(base, RPT)], out_hbm.at[c, pl.ds(base, RPT)])


@functools.partial(
    pl.kernel,
    out_type=jax.ShapeDtypeStruct((NC, NP, DO), jnp.float32),
    mesh=_MESH(),
    compiler_params=_SC_PARAMS,
    scratch_types=[
        pltpu.VMEM((NB, EB), jnp.int32),
        pltpu.VMEM((NB, EB), jnp.int32),
        pltpu.VMEM((2, EB, DO), jnp.float32),
        pltpu.VMEM_SHARED((NP, DO), jnp.float32),
        pltpu.SemaphoreType.DMA((2,)),
    ],
)
def _scat_kernel(src_hbm, dst_hbm, h2_hbm, zeros_hbm, out_hbm,
                 src_v, dst_v, rows_v, agg_sh, sem):
    c = lax.axis_index("c")
    s = lax.axis_index("s")
    wid = s * NC + c
    base = s * RPT
    pltpu.sync_copy(zeros_hbm.at[pl.ds(base, RPT)], agg_sh.at[pl.ds(base, RPT)])
    pltpu.sync_copy(src_hbm.at[wid], src_v)
    pltpu.sync_copy(dst_hbm.at[wid], dst_v)
    plsc.subcore_barrier()

    pltpu.async_copy(h2_hbm.at[src_v.at[0]], rows_v.at[0], sem.at[0])
    pltpu.async_copy(h2_hbm.at[src_v.at[1]], rows_v.at[1], sem.at[1])

    def outer(g, carry):
        for j in range(2):
            b = g * 2 + j
            pltpu.make_async_copy(h2_hbm.at[src_v.at[0]], rows_v.at[j],
                                  sem.at[j]).wait()
            pltpu.sync_copy(rows_v.at[j], agg_sh.at[dst_v.at[b]], add=True)
            bn = lax.rem(b + 2, NB)
            pltpu.async_copy(h2_hbm.at[src_v.at[bn]], rows_v.at[j], sem.at[j])
        return carry

    lax.fori_loop(0, NB // 2, outer, 0)
    for j in range(2):
        pltpu.make_async_copy(h2_hbm.at[src_v.at[0]], rows_v.at[j],
                              sem.at[j]).wait()
    plsc.subcore_barrier()
    pltpu.sync_copy(agg_sh.at[pl.ds(base, RPT)], out_hbm.at[c, pl.ds(base, RPT)])


_BLK = 2048


def _mm_body(x_ref, wg_ref, w1_ref, w3_ref, c0_ref, c1_ref, h2_ref):
    h = lax.dot_general(x_ref[...], wg_ref[...], (((1,), (1,)), ((), ())),
                        preferred_element_type=jnp.float32)
    h = lax.dot_general(h, w1_ref[...], (((1,), (1,)), ((), ())),
                        preferred_element_type=jnp.float32)
    h = lax.dot_general(h, w3_ref[...], (((1,), (1,)), ((), ())),
                        preferred_element_type=jnp.float32)
    cnt = c0_ref[...][:, 0:1] + c1_ref[...][:, 0:1]
    h2_ref[...] = h * lax.rsqrt(cnt + 1.0)


_mm_call = pl.pallas_call(
    _mm_body,
    grid=(NP // _BLK,),
    in_specs=[
        pl.BlockSpec((_BLK, DI), lambda i: (i, 0)),
        pl.BlockSpec((DI, DI), lambda i: (0, 0)),
        pl.BlockSpec((DO, DI), lambda i: (0, 0)),
        pl.BlockSpec((DO, DO), lambda i: (0, 0)),
        pl.BlockSpec((_BLK, 8), lambda i: (i, 0)),
        pl.BlockSpec((_BLK, 8), lambda i: (i, 0)),
    ],
    out_specs=pl.BlockSpec((_BLK, DO), lambda i: (i, 0)),
    out_shape=jax.ShapeDtypeStruct((NP, DO), jnp.float32),
)


def _epi_body(a0_ref, a1_ref, h2_ref, c0_ref, c1_ref,
              w1_ref, w3_ref, bg_ref, b1_ref, b3_ref, out_ref):
    cr = lax.dot_general(bg_ref[...], w1_ref[...], (((1,), (1,)), ((), ())),
                         preferred_element_type=jnp.float32) + b1_ref[...]
    cr = lax.dot_general(cr, w3_ref[...], (((1,), (1,)), ((), ())),
                         preferred_element_type=jnp.float32) + b3_ref[...]
    deg = c0_ref[...][:, 0:1] + c1_ref[...][:, 0:1] + 1.0
    agg = a0_ref[...] + a1_ref[...] + h2_ref[...]
    out_ref[...] = agg * lax.rsqrt(deg) + cr


_epi_call = pl.pallas_call(
    _epi_body,
    grid=(NP // _BLK,),
    in_specs=[
        pl.BlockSpec((_BLK, DO), lambda i: (i, 0)),
        pl.BlockSpec((_BLK, DO), lambda i: (i, 0)),
        pl.BlockSpec((_BLK, DO), lambda i: (i, 0)),
        pl.BlockSpec((_BLK, 8), lambda i: (i, 0)),
        pl.BlockSpec((_BLK, 8), lambda i: (i, 0)),
        pl.BlockSpec((DO, DI), lambda i: (0, 0)),
        pl.BlockSpec((DO, DO), lambda i: (0, 0)),
        pl.BlockSpec((1, DI), lambda i: (0, 0)),
        pl.BlockSpec((1, DO), lambda i: (0, 0)),
        pl.BlockSpec((1, DO), lambda i: (0, 0)),
    ],
    out_specs=pl.BlockSpec((_BLK, DO), lambda i: (i, 0)),
    out_shape=jax.ShapeDtypeStruct((NP, DO), jnp.float32),
)


def kernel(x, edge_index, batch, W_gcn, b_gcn, W1, b1, W3, b3):
    src = edge_index[0]
    dst = edge_index[1]
    pad = jnp.full((EP - E,), N, dtype=jnp.int32)
    src_r = jnp.concatenate([src, pad]).reshape(NW, NB, EB)
    dst_r = jnp.concatenate([dst, pad]).reshape(NW, NB, EB)
    x_pad = jnp.zeros((NP, DI), jnp.float32).at[:N].set(x)
    ones8 = jnp.ones((EB, 8), jnp.float32)
    zeros8 = jnp.zeros((NP, 8), jnp.float32)
    zeros64 = jnp.zeros((NP, DO), jnp.float32)

    cnts = _cnt_kernel(dst_r, ones8, zeros8)
    h2 = _mm_call(x_pad, W_gcn, W1, W3, cnts[0], cnts[1])
    aggs = _scat_kernel(src_r, dst_r, h2, zeros64)
    out = _epi_call(aggs[0], aggs[1], h2, cnts[0], cnts[1], W1, W3,
                    b_gcn.reshape(1, DI), b1.reshape(1, DO), b3.reshape(1, DO))
    return out[:N]


# speedup vs baseline: 39.1510x; 1.9415x over previous
import functools

import jax
import jax.numpy as jnp
from jax import lax
from jax.experimental import pallas as pl
from jax.experimental.pallas import tpu as pltpu
from jax.experimental.pallas import tpu_sc as plsc

N = 10000
NP = 10240
DI = 128
DO = 64
E = 320000
NC = 2
NS = 16
NW = NC * NS
EB = 128
NB = 80
EP = NW * NB * EB
RPT = NP // NS

_MESH = functools.partial(
    plsc.VectorSubcoreMesh,
    core_axis_name="c", subcore_axis_name="s", num_cores=NC, num_subcores=NS,
)
_SC_PARAMS = pltpu.CompilerParams(use_tc_tiling_on_sc=False)


@functools.partial(
    pl.kernel,
    out_type=jax.ShapeDtypeStruct((NC, NP, 8), jnp.float32),
    mesh=_MESH(),
    compiler_params=_SC_PARAMS,
    scratch_types=[
        pltpu.VMEM((NB, EB), jnp.int32),
        pltpu.VMEM((EB, 8), jnp.float32),
        pltpu.VMEM_SHARED((NP, 8), jnp.float32),
    ],
)
def _cnt_kernel(dst_hbm, ones_hbm, zeros_hbm, out_hbm, dst_v, ones_v, cnt_sh):
    c = lax.axis_index("c")
    s = lax.axis_index("s")
    wid = s * NC + c
    base = s * RPT
    pltpu.sync_copy(zeros_hbm.at[pl.ds(base, RPT)], cnt_sh.at[pl.ds(base, RPT)])
    pltpu.sync_copy(ones_hbm, ones_v)
    pltpu.sync_copy(dst_hbm.at[wid], dst_v)
    plsc.subcore_barrier()

    def body(b, carry):
        pltpu.sync_copy(ones_v, cnt_sh.at[dst_v.at[b]], add=True)
        return carry

    lax.fori_loop(0, NB, body, 0)
    plsc.subcore_barrier()
    pltpu.sync_copy(cnt_sh.at[pl.ds(base, RPT)], out_hbm.at[c, pl.ds(base, RPT)])


@functools.partial(
    pl.kernel,
    out_type=jax.ShapeDtypeStruct((NC, NP, DO), jnp.float32),
    mesh=_MESH(),
    compiler_params=_SC_PARAMS,
    scratch_types=[
        pltpu.VMEM((NB, EB), jnp.int32),
        pltpu.VMEM((NB, EB), jnp.int32),
        pltpu.VMEM((2, EB, DO), jnp.float32),
        pltpu.VMEM_SHARED((NP, DO), jnp.float32),
        pltpu.VMEM_SHARED((NP, DO), jnp.float32),
        pltpu.SemaphoreType.DMA((2,)),
    ],
)
def _scat_kernel(src_hbm, dst_hbm, h2_hbm, zeros_hbm, out_hbm,
                 src_v, dst_v, rows_v, agg_sh, h2_sh, sem):
    c = lax.axis_index("c")
    s = lax.axis_index("s")
    wid = s * NC + c
    base = s * RPT
    pltpu.sync_copy(zeros_hbm.at[pl.ds(base, RPT)], agg_sh.at[pl.ds(base, RPT)])
    pltpu.sync_copy(h2_hbm.at[pl.ds(base, RPT)], h2_sh.at[pl.ds(base, RPT)])
    pltpu.sync_copy(src_hbm.at[wid], src_v)
    pltpu.sync_copy(dst_hbm.at[wid], dst_v)
    plsc.subcore_barrier()

    pltpu.async_copy(h2_sh.at[src_v.at[0]], rows_v.at[0], sem.at[0])
    pltpu.async_copy(h2_sh.at[src_v.at[1]], rows_v.at[1], sem.at[1])

    def outer(g, carry):
        for j in range(2):
            b = g * 2 + j
            pltpu.make_async_copy(h2_hbm.at[src_v.at[0]], rows_v.at[j],
                                  sem.at[j]).wait()
            pltpu.sync_copy(rows_v.at[j], agg_sh.at[dst_v.at[b]], add=True)
            bn = lax.rem(b + 2, NB)
            pltpu.async_copy(h2_sh.at[src_v.at[bn]], rows_v.at[j], sem.at[j])
        return carry

    lax.fori_loop(0, NB // 2, outer, 0)
    for j in range(2):
        pltpu.make_async_copy(h2_hbm.at[src_v.at[0]], rows_v.at[j],
                              sem.at[j]).wait()
    plsc.subcore_barrier()
    pltpu.sync_copy(agg_sh.at[pl.ds(base, RPT)], out_hbm.at[c, pl.ds(base, RPT)])


_BLK = 2048


def _mm_body(x_ref, wg_ref, w1_ref, w3_ref, c0_ref, c1_ref, h2_ref):
    h = lax.dot_general(x_ref[...], wg_ref[...], (((1,), (1,)), ((), ())),
                        preferred_element_type=jnp.float32)
    h = lax.dot_general(h, w1_ref[...], (((1,), (1,)), ((), ())),
                        preferred_element_type=jnp.float32)
    h = lax.dot_general(h, w3_ref[...], (((1,), (1,)), ((), ())),
                        preferred_element_type=jnp.float32)
    cnt = c0_ref[...][:, 0:1] + c1_ref[...][:, 0:1]
    h2_ref[...] = h * lax.rsqrt(cnt + 1.0)


_mm_call = pl.pallas_call(
    _mm_body,
    grid=(NP // _BLK,),
    in_specs=[
        pl.BlockSpec((_BLK, DI), lambda i: (i, 0)),
        pl.BlockSpec((DI, DI), lambda i: (0, 0)),
        pl.BlockSpec((DO, DI), lambda i: (0, 0)),
        pl.BlockSpec((DO, DO), lambda i: (0, 0)),
        pl.BlockSpec((_BLK, 8), lambda i: (i, 0)),
        pl.BlockSpec((_BLK, 8), lambda i: (i, 0)),
    ],
    out_specs=pl.BlockSpec((_BLK, DO), lambda i: (i, 0)),
    out_shape=jax.ShapeDtypeStruct((NP, DO), jnp.float32),
)


def _epi_body(a0_ref, a1_ref, h2_ref, c0_ref, c1_ref,
              w1_ref, w3_ref, bg_ref, b1_ref, b3_ref, out_ref):
    cr = lax.dot_general(bg_ref[...], w1_ref[...], (((1,), (1,)), ((), ())),
                         preferred_element_type=jnp.float32) + b1_ref[...]
    cr = lax.dot_general(cr, w3_ref[...], (((1,), (1,)), ((), ())),
                         preferred_element_type=jnp.float32) + b3_ref[...]
    deg = c0_ref[...][:, 0:1] + c1_ref[...][:, 0:1] + 1.0
    agg = a0_ref[...] + a1_ref[...] + h2_ref[...]
    out_ref[...] = agg * lax.rsqrt(deg) + cr


_epi_call = pl.pallas_call(
    _epi_body,
    grid=(NP // _BLK,),
    in_specs=[
        pl.BlockSpec((_BLK, DO), lambda i: (i, 0)),
        pl.BlockSpec((_BLK, DO), lambda i: (i, 0)),
        pl.BlockSpec((_BLK, DO), lambda i: (i, 0)),
        pl.BlockSpec((_BLK, 8), lambda i: (i, 0)),
        pl.BlockSpec((_BLK, 8), lambda i: (i, 0)),
        pl.BlockSpec((DO, DI), lambda i: (0, 0)),
        pl.BlockSpec((DO, DO), lambda i: (0, 0)),
        pl.BlockSpec((1, DI), lambda i: (0, 0)),
        pl.BlockSpec((1, DO), lambda i: (0, 0)),
        pl.BlockSpec((1, DO), lambda i: (0, 0)),
    ],
    out_specs=pl.BlockSpec((_BLK, DO), lambda i: (i, 0)),
    out_shape=jax.ShapeDtypeStruct((NP, DO), jnp.float32),
)


def kernel(x, edge_index, batch, W_gcn, b_gcn, W1, b1, W3, b3):
    src = edge_index[0]
    dst = edge_index[1]
    pad = jnp.full((EP - E,), N, dtype=jnp.int32)
    src_r = jnp.concatenate([src, pad]).reshape(NW, NB, EB)
    dst_r = jnp.concatenate([dst, pad]).reshape(NW, NB, EB)
    x_pad = jnp.zeros((NP, DI), jnp.float32).at[:N].set(x)
    ones8 = jnp.ones((EB, 8), jnp.float32)
    zeros8 = jnp.zeros((NP, 8), jnp.float32)
    zeros64 = jnp.zeros((NP, DO), jnp.float32)

    cnts = _cnt_kernel(dst_r, ones8, zeros8)
    h2 = _mm_call(x_pad, W_gcn, W1, W3, cnts[0], cnts[1])
    aggs = _scat_kernel(src_r, dst_r, h2, zeros64)
    out = _epi_call(aggs[0], aggs[1], h2, cnts[0], cnts[1], W1, W3,
                    b_gcn.reshape(1, DI), b1.reshape(1, DO), b3.reshape(1, DO))
    return out[:N]


# speedup vs baseline: 46.7137x; 1.1932x over previous
import functools

import jax
import jax.numpy as jnp
from jax import lax
from jax.experimental import pallas as pl
from jax.experimental.pallas import tpu as pltpu
from jax.experimental.pallas import tpu_sc as plsc

N = 10000
DI = 128
DO = 64
E = 320000
NC = 2
NS = 16
NW = NC * NS
EB = 128
NBT = E // EB
NBMAX = NBT // NW + 1
RPT = N // NS

_MESH = functools.partial(
    plsc.VectorSubcoreMesh,
    core_axis_name="c", subcore_axis_name="s", num_cores=NC, num_subcores=NS,
)
_SC_PARAMS = pltpu.CompilerParams(use_tc_tiling_on_sc=False)


def _tile_ids():
    c = lax.axis_index("c")
    s = lax.axis_index("s")
    wid = s * NC + c
    b0 = (wid * NBT) // NW
    nb = ((wid + 1) * NBT) // NW - b0
    return c, s, wid, b0, nb


@functools.partial(
    pl.kernel,
    out_type=jax.ShapeDtypeStruct((NC, N, 8), jnp.float32),
    mesh=_MESH(),
    compiler_params=_SC_PARAMS,
    scratch_types=[
        pltpu.VMEM((NBMAX, EB), jnp.int32),
        pltpu.VMEM((EB, 8), jnp.float32),
        pltpu.VMEM_SHARED((N, 8), jnp.float32),
    ],
)
def _cnt_kernel(ei_hbm, ones_hbm, zeros_hbm, out_hbm, dst_v, ones_v, cnt_sh):
    c, s, wid, b0, nb = _tile_ids()
    base = s * RPT
    pltpu.sync_copy(zeros_hbm.at[pl.ds(base, RPT)], cnt_sh.at[pl.ds(base, RPT)])
    pltpu.sync_copy(ones_hbm, ones_v)
    pltpu.sync_copy(ei_hbm.at[1, pl.ds(b0, NBMAX)], dst_v)
    plsc.subcore_barrier()

    def body(b, carry):
        pltpu.sync_copy(ones_v, cnt_sh.at[dst_v.at[b]], add=True)
        return carry

    lax.fori_loop(0, nb, body, 0)
    plsc.subcore_barrier()
    pltpu.sync_copy(cnt_sh.at[pl.ds(base, RPT)], out_hbm.at[c, pl.ds(base, RPT)])


@functools.partial(
    pl.kernel,
    out_type=jax.ShapeDtypeStruct((NC, N, DO), jnp.float32),
    mesh=_MESH(),
    compiler_params=_SC_PARAMS,
    scratch_types=[
        pltpu.VMEM((NBMAX, EB), jnp.int32),
        pltpu.VMEM((NBMAX, EB), jnp.int32),
        pltpu.VMEM((2, EB, DO), jnp.float32),
        pltpu.VMEM_SHARED((N, DO), jnp.float32),
        pltpu.VMEM_SHARED((N, DO), jnp.float32),
        pltpu.SemaphoreType.DMA((2,)),
    ],
)
def _scat_kernel(ei_hbm, h2_hbm, zeros_hbm, out_hbm,
                 src_v, dst_v, rows_v, agg_sh, h2_sh, sem):
    c, s, wid, b0, nb = _tile_ids()
    base = s * RPT
    pltpu.sync_copy(zeros_hbm.at[pl.ds(base, RPT)], agg_sh.at[pl.ds(base, RPT)])
    pltpu.sync_copy(h2_hbm.at[pl.ds(base, RPT)], h2_sh.at[pl.ds(base, RPT)])
    pltpu.sync_copy(ei_hbm.at[0, pl.ds(b0, NBMAX)], src_v)
    pltpu.sync_copy(ei_hbm.at[1, pl.ds(b0, NBMAX)], dst_v)
    plsc.subcore_barrier()

    pltpu.async_copy(h2_sh.at[src_v.at[0]], rows_v.at[0], sem.at[0])
    pltpu.async_copy(h2_sh.at[src_v.at[1]], rows_v.at[1], sem.at[1])

    def body(b, carry):
        j = lax.rem(b, 2)
        pltpu.make_async_copy(h2_hbm.at[src_v.at[0]], rows_v.at[j],
                              sem.at[j]).wait()
        pltpu.sync_copy(rows_v.at[j], agg_sh.at[dst_v.at[b]], add=True)
        pltpu.async_copy(h2_sh.at[src_v.at[lax.rem(b + 2, nb)]],
                         rows_v.at[j], sem.at[j])
        return carry

    lax.fori_loop(0, nb, body, 0)
    for j in range(2):
        pltpu.make_async_copy(h2_hbm.at[src_v.at[0]], rows_v.at[j],
                              sem.at[j]).wait()
    plsc.subcore_barrier()
    pltpu.sync_copy(agg_sh.at[pl.ds(base, RPT)], out_hbm.at[c, pl.ds(base, RPT)])


_BLK = 2000


def _mm_body(x_ref, wg_ref, w1_ref, w3_ref, c_ref, h2_ref):
    h = lax.dot_general(x_ref[...], wg_ref[...], (((1,), (1,)), ((), ())),
                        preferred_element_type=jnp.float32)
    h = lax.dot_general(h, w1_ref[...], (((1,), (1,)), ((), ())),
                        preferred_element_type=jnp.float32)
    h = lax.dot_general(h, w3_ref[...], (((1,), (1,)), ((), ())),
                        preferred_element_type=jnp.float32)
    cn = c_ref[...]
    cnt = cn[0, :, 0:1] + cn[1, :, 0:1]
    h2_ref[...] = h * lax.rsqrt(cnt + 1.0)


_mm_call = pl.pallas_call(
    _mm_body,
    grid=(N // _BLK,),
    in_specs=[
        pl.BlockSpec((_BLK, DI), lambda i: (i, 0)),
        pl.BlockSpec((DI, DI), lambda i: (0, 0)),
        pl.BlockSpec((DO, DI), lambda i: (0, 0)),
        pl.BlockSpec((DO, DO), lambda i: (0, 0)),
        pl.BlockSpec((NC, _BLK, 8), lambda i: (0, i, 0)),
    ],
    out_specs=pl.BlockSpec((_BLK, DO), lambda i: (i, 0)),
    out_shape=jax.ShapeDtypeStruct((N, DO), jnp.float32),
)


def _epi_body(a_ref, h2_ref, c_ref, w1_ref, w3_ref, bg_ref, b1_ref, b3_ref,
              out_ref):
    cr = lax.dot_general(bg_ref[...], w1_ref[...], (((1,), (1,)), ((), ())),
                         preferred_element_type=jnp.float32) + b1_ref[...]
    cr = lax.dot_general(cr, w3_ref[...], (((1,), (1,)), ((), ())),
                         preferred_element_type=jnp.float32) + b3_ref[...]
    cn = c_ref[...]
    deg = cn[0, :, 0:1] + cn[1, :, 0:1] + 1.0
    a = a_ref[...]
    agg = a[0] + a[1] + h2_ref[...]
    out_ref[...] = agg * lax.rsqrt(deg) + cr


_epi_call = pl.pallas_call(
    _epi_body,
    grid=(N // _BLK,),
    in_specs=[
        pl.BlockSpec((NC, _BLK, DO), lambda i: (0, i, 0)),
        pl.BlockSpec((_BLK, DO), lambda i: (i, 0)),
        pl.BlockSpec((NC, _BLK, 8), lambda i: (0, i, 0)),
        pl.BlockSpec((DO, DI), lambda i: (0, 0)),
        pl.BlockSpec((DO, DO), lambda i: (0, 0)),
        pl.BlockSpec((1, DI), lambda i: (0, 0)),
        pl.BlockSpec((1, DO), lambda i: (0, 0)),
        pl.BlockSpec((1, DO), lambda i: (0, 0)),
    ],
    out_specs=pl.BlockSpec((_BLK, DO), lambda i: (i, 0)),
    out_shape=jax.ShapeDtypeStruct((N, DO), jnp.float32),
)


def kernel(x, edge_index, batch, W_gcn, b_gcn, W1, b1, W3, b3):
    ei = edge_index.reshape(2, NBT, EB)
    ones8 = jnp.ones((EB, 8), jnp.float32)
    zeros8 = jnp.zeros((N, 8), jnp.float32)
    zeros64 = jnp.zeros((N, DO), jnp.float32)

    cnts = _cnt_kernel(ei, ones8, zeros8)
    h2 = _mm_call(x, W_gcn, W1, W3, cnts)
    aggs = _scat_kernel(ei, h2, zeros64)
    return _epi_call(aggs, h2, cnts, W1, W3,
                     b_gcn.reshape(1, DI), b1.reshape(1, DO),
                     b3.reshape(1, DO))


# speedup vs baseline: 51.6359x; 1.1054x over previous
import functools

import jax
import jax.numpy as jnp
from jax import lax
from jax.experimental import pallas as pl
from jax.experimental.pallas import tpu as pltpu
from jax.experimental.pallas import tpu_sc as plsc

N = 10000
DI = 128
DO = 64
E = 320000
NC = 2
NS = 16
NW = NC * NS
EB = 128
NBT = E // EB
NBMAX = NBT // NW + 1
RPT = N // NS

_MESH = functools.partial(
    plsc.VectorSubcoreMesh,
    core_axis_name="c", subcore_axis_name="s", num_cores=NC, num_subcores=NS,
)
_SC_PARAMS = pltpu.CompilerParams(use_tc_tiling_on_sc=False)


def _tile_ids():
    c = lax.axis_index("c")
    s = lax.axis_index("s")
    wid = s * NC + c
    b0 = (wid * NBT) // NW
    nb = ((wid + 1) * NBT) // NW - b0
    return c, s, wid, b0, nb


@functools.partial(
    pl.kernel,
    out_type=jax.ShapeDtypeStruct((NC, N, 8), jnp.float32),
    mesh=_MESH(),
    compiler_params=_SC_PARAMS,
    scratch_types=[
        pltpu.VMEM((NBMAX, EB), jnp.int32),
        pltpu.VMEM((EB, 8), jnp.float32),
        pltpu.VMEM_SHARED((N, 8), jnp.float32),
    ],
)
def _cnt_kernel(ei_hbm, ones_hbm, zeros_hbm, out_hbm, dst_v, ones_v, cnt_sh):
    c, s, wid, b0, nb = _tile_ids()
    base = s * RPT
    pltpu.sync_copy(zeros_hbm.at[pl.ds(base, RPT)], cnt_sh.at[pl.ds(base, RPT)])
    pltpu.sync_copy(ones_hbm, ones_v)
    pltpu.sync_copy(ei_hbm.at[1, pl.ds(b0, NBMAX)], dst_v)
    plsc.subcore_barrier()

    def body(b, carry):
        pltpu.sync_copy(ones_v, cnt_sh.at[dst_v.at[b]], add=True)
        return carry

    lax.fori_loop(0, nb, body, 0)
    plsc.subcore_barrier()
    pltpu.sync_copy(cnt_sh.at[pl.ds(base, RPT)], out_hbm.at[c, pl.ds(base, RPT)])


@functools.partial(
    pl.kernel,
    out_type=jax.ShapeDtypeStruct((NC, N, DO), jnp.float32),
    mesh=_MESH(),
    compiler_params=_SC_PARAMS,
    scratch_types=[
        pltpu.VMEM((NBMAX, EB), jnp.int32),
        pltpu.VMEM((NBMAX, EB), jnp.int32),
        pltpu.VMEM((3, EB, DO), jnp.float32),
        pltpu.VMEM_SHARED((N, DO), jnp.float32),
        pltpu.VMEM_SHARED((N, DO), jnp.float32),
        pltpu.SemaphoreType.DMA((3,)),
        pltpu.SemaphoreType.DMA((3,)),
    ],
)
def _scat_kernel(ei_hbm, h2_hbm, zeros_hbm, out_hbm,
                 src_v, dst_v, rows_v, agg_sh, h2_sh, semg, sems):
    c, s, wid, b0, nb = _tile_ids()
    base = s * RPT
    pltpu.sync_copy(zeros_hbm.at[pl.ds(base, RPT)], agg_sh.at[pl.ds(base, RPT)])
    pltpu.sync_copy(h2_hbm.at[pl.ds(base, RPT)], h2_sh.at[pl.ds(base, RPT)])
    pltpu.sync_copy(ei_hbm.at[0, pl.ds(b0, NBMAX)], src_v)
    pltpu.sync_copy(ei_hbm.at[1, pl.ds(b0, NBMAX)], dst_v)
    plsc.subcore_barrier()

    for j in range(3):
        pltpu.async_copy(h2_sh.at[src_v.at[j]], rows_v.at[j], semg.at[j])

    def body(b, carry):
        j = lax.rem(b, 3)
        pltpu.make_async_copy(h2_hbm.at[src_v.at[0]], rows_v.at[j],
                              semg.at[j]).wait()
        pltpu.async_copy(rows_v.at[j], agg_sh.at[dst_v.at[b]], sems.at[j],
                         add=True)
        j2 = lax.rem(b + 2, 3)

        @pl.when(b >= 1)
        def _():
            pltpu.make_async_copy(rows_v.at[j2], agg_sh.at[dst_v.at[0]],
                                  sems.at[j2]).wait()
            pltpu.async_copy(h2_sh.at[src_v.at[lax.rem(b + 2, nb)]],
                             rows_v.at[j2], semg.at[j2])

        return carry

    lax.fori_loop(0, nb, body, 0)
    js = lax.rem(nb + 2, 3)
    pltpu.make_async_copy(rows_v.at[js], agg_sh.at[dst_v.at[0]],
                          sems.at[js]).wait()
    for d in range(2):
        jg = lax.rem(nb + d, 3)
        pltpu.make_async_copy(h2_hbm.at[src_v.at[0]], rows_v.at[jg],
                              semg.at[jg]).wait()
    plsc.subcore_barrier()
    pltpu.sync_copy(agg_sh.at[pl.ds(base, RPT)], out_hbm.at[c, pl.ds(base, RPT)])


_BLK = 2000


def _mm_body(x_ref, wg_ref, w1_ref, w3_ref, c_ref, h2_ref):
    h = lax.dot_general(x_ref[...], wg_ref[...], (((1,), (1,)), ((), ())),
                        preferred_element_type=jnp.float32)
    h = lax.dot_general(h, w1_ref[...], (((1,), (1,)), ((), ())),
                        preferred_element_type=jnp.float32)
    h = lax.dot_general(h, w3_ref[...], (((1,), (1,)), ((), ())),
                        preferred_element_type=jnp.float32)
    cn = c_ref[...]
    cnt = cn[0, :, 0:1] + cn[1, :, 0:1]
    h2_ref[...] = h * lax.rsqrt(cnt + 1.0)


_mm_call = pl.pallas_call(
    _mm_body,
    grid=(N // _BLK,),
    in_specs=[
        pl.BlockSpec((_BLK, DI), lambda i: (i, 0)),
        pl.BlockSpec((DI, DI), lambda i: (0, 0)),
        pl.BlockSpec((DO, DI), lambda i: (0, 0)),
        pl.BlockSpec((DO, DO), lambda i: (0, 0)),
        pl.BlockSpec((NC, _BLK, 8), lambda i: (0, i, 0)),
    ],
    out_specs=pl.BlockSpec((_BLK, DO), lambda i: (i, 0)),
    out_shape=jax.ShapeDtypeStruct((N, DO), jnp.float32),
)


def _epi_body(a_ref, h2_ref, c_ref, w1_ref, w3_ref, bg_ref, b1_ref, b3_ref,
              out_ref):
    cr = lax.dot_general(bg_ref[...], w1_ref[...], (((1,), (1,)), ((), ())),
                         preferred_element_type=jnp.float32) + b1_ref[...]
    cr = lax.dot_general(cr, w3_ref[...], (((1,), (1,)), ((), ())),
                         preferred_element_type=jnp.float32) + b3_ref[...]
    cn = c_ref[...]
    deg = cn[0, :, 0:1] + cn[1, :, 0:1] + 1.0
    a = a_ref[...]
    agg = a[0] + a[1] + h2_ref[...]
    out_ref[...] = agg * lax.rsqrt(deg) + cr


_epi_call = pl.pallas_call(
    _epi_body,
    grid=(N // _BLK,),
    in_specs=[
        pl.BlockSpec((NC, _BLK, DO), lambda i: (0, i, 0)),
        pl.BlockSpec((_BLK, DO), lambda i: (i, 0)),
        pl.BlockSpec((NC, _BLK, 8), lambda i: (0, i, 0)),
        pl.BlockSpec((DO, DI), lambda i: (0, 0)),
        pl.BlockSpec((DO, DO), lambda i: (0, 0)),
        pl.BlockSpec((1, DI), lambda i: (0, 0)),
        pl.BlockSpec((1, DO), lambda i: (0, 0)),
        pl.BlockSpec((1, DO), lambda i: (0, 0)),
    ],
    out_specs=pl.BlockSpec((_BLK, DO), lambda i: (i, 0)),
    out_shape=jax.ShapeDtypeStruct((N, DO), jnp.float32),
)


def kernel(x, edge_index, batch, W_gcn, b_gcn, W1, b1, W3, b3):
    ei = edge_index.reshape(2, NBT, EB)
    ones8 = jnp.ones((EB, 8), jnp.float32)
    zeros8 = jnp.zeros((N, 8), jnp.float32)
    zeros64 = jnp.zeros((N, DO), jnp.float32)

    cnts = _cnt_kernel(ei, ones8, zeros8)
    h2 = _mm_call(x, W_gcn, W1, W3, cnts)
    aggs = _scat_kernel(ei, h2, zeros64)
    return _epi_call(aggs, h2, cnts, W1, W3,
                     b_gcn.reshape(1, DI), b1.reshape(1, DO),
                     b3.reshape(1, DO))


# speedup vs baseline: 56.4207x; 1.0927x over previous
import functools

import jax
import jax.numpy as jnp
from jax import lax
from jax.experimental import pallas as pl
from jax.experimental.pallas import tpu as pltpu
from jax.experimental.pallas import tpu_sc as plsc

N = 10000
DI = 128
DO = 64
E = 320000
NC = 2
NS = 16
NW = NC * NS
EB = 128
NBT = E // EB
NBMAX = NBT // NW + 1
RPT = N // NS

_MESH = functools.partial(
    plsc.VectorSubcoreMesh,
    core_axis_name="c", subcore_axis_name="s", num_cores=NC, num_subcores=NS,
)
_SC_PARAMS = pltpu.CompilerParams(use_tc_tiling_on_sc=False)


def _tile_ids():
    c = lax.axis_index("c")
    s = lax.axis_index("s")
    wid = s * NC + c
    b0 = (wid * NBT) // NW
    nb = ((wid + 1) * NBT) // NW - b0
    return c, s, wid, b0, nb


@functools.partial(
    pl.kernel,
    out_type=jax.ShapeDtypeStruct((NC, N), jnp.float32),
    mesh=_MESH(),
    compiler_params=_SC_PARAMS,
    scratch_types=[
        pltpu.VMEM((NBMAX, EB), jnp.int32),
        pltpu.VMEM((EB,), jnp.float32),
        pltpu.VMEM_SHARED((N,), jnp.float32),
    ],
)
def _cnt_kernel(ei_hbm, ones_hbm, zeros_hbm, out_hbm, dst_v, ones_v, cnt_sh):
    c, s, wid, b0, nb = _tile_ids()
    r0 = lax.div(s * RPT, 8) * 8
    pltpu.sync_copy(zeros_hbm.at[pl.ds(r0, RPT + 7)],
                    cnt_sh.at[pl.ds(r0, RPT + 7)])
    pltpu.sync_copy(ones_hbm, ones_v)
    pltpu.sync_copy(ei_hbm.at[1, pl.ds(b0, NBMAX)], dst_v)
    plsc.subcore_barrier()

    def body(b, carry):
        pltpu.sync_copy(ones_v, cnt_sh.at[dst_v.at[b]], add=True)
        return carry

    lax.fori_loop(0, nb, body, 0)
    plsc.subcore_barrier()
    pltpu.sync_copy(cnt_sh.at[pl.ds(r0, RPT + 7)],
                    out_hbm.at[c, pl.ds(r0, RPT + 7)])


@functools.partial(
    pl.kernel,
    out_type=jax.ShapeDtypeStruct((NC, N, DO), jnp.float32),
    mesh=_MESH(),
    compiler_params=_SC_PARAMS,
    scratch_types=[
        pltpu.VMEM((NBMAX, EB), jnp.int32),
        pltpu.VMEM((NBMAX, EB), jnp.int32),
        pltpu.VMEM((3, EB, DO), jnp.float32),
        pltpu.VMEM_SHARED((N, DO), jnp.float32),
        pltpu.VMEM_SHARED((N, DO), jnp.float32),
        pltpu.SemaphoreType.DMA((3,)),
        pltpu.SemaphoreType.DMA((3,)),
    ],
)
def _scat_kernel(ei_hbm, h2_hbm, zeros_hbm, out_hbm,
                 src_v, dst_v, rows_v, agg_sh, h2_sh, semg, sems):
    c, s, wid, b0, nb = _tile_ids()
    base = s * RPT
    pltpu.sync_copy(zeros_hbm.at[pl.ds(base, RPT)], agg_sh.at[pl.ds(base, RPT)])
    pltpu.sync_copy(h2_hbm.at[pl.ds(base, RPT)], h2_sh.at[pl.ds(base, RPT)])
    pltpu.sync_copy(ei_hbm.at[0, pl.ds(b0, NBMAX)], src_v)
    pltpu.sync_copy(ei_hbm.at[1, pl.ds(b0, NBMAX)], dst_v)
    plsc.subcore_barrier()

    for j in range(3):
        pltpu.async_copy(h2_sh.at[src_v.at[j]], rows_v.at[j], semg.at[j])

    def body(b, carry):
        j = lax.rem(b, 3)
        pltpu.make_async_copy(h2_hbm.at[src_v.at[0]], rows_v.at[j],
                              semg.at[j]).wait()
        pltpu.async_copy(rows_v.at[j], agg_sh.at[dst_v.at[b]], sems.at[j],
                         add=True)
        j2 = lax.rem(b + 2, 3)

        @pl.when(b >= 1)
        def _():
            pltpu.make_async_copy(rows_v.at[j2], agg_sh.at[dst_v.at[0]],
                                  sems.at[j2]).wait()
            pltpu.async_copy(h2_sh.at[src_v.at[lax.rem(b + 2, nb)]],
                             rows_v.at[j2], semg.at[j2])

        return carry

    lax.fori_loop(0, nb, body, 0)
    js = lax.rem(nb + 2, 3)
    pltpu.make_async_copy(rows_v.at[js], agg_sh.at[dst_v.at[0]],
                          sems.at[js]).wait()
    for d in range(2):
        jg = lax.rem(nb + d, 3)
        pltpu.make_async_copy(h2_hbm.at[src_v.at[0]], rows_v.at[jg],
                              semg.at[jg]).wait()
    plsc.subcore_barrier()
    pltpu.sync_copy(agg_sh.at[pl.ds(base, RPT)], out_hbm.at[c, pl.ds(base, RPT)])


def _mm_body(x_ref, wg_ref, w1_ref, w3_ref, c_ref, h2_ref):
    h = lax.dot_general(x_ref[...], wg_ref[...], (((1,), (1,)), ((), ())),
                        preferred_element_type=jnp.float32)
    h = lax.dot_general(h, w1_ref[...], (((1,), (1,)), ((), ())),
                        preferred_element_type=jnp.float32)
    h = lax.dot_general(h, w3_ref[...], (((1,), (1,)), ((), ())),
                        preferred_element_type=jnp.float32)
    cn = c_ref[...]
    cnt = (cn[0] + cn[1])[:, None]
    h2_ref[...] = h * lax.rsqrt(cnt + 1.0)


_mm_call = pl.pallas_call(
    _mm_body,
    out_shape=jax.ShapeDtypeStruct((N, DO), jnp.float32),
)


def _epi_body(a_ref, h2_ref, c_ref, w1_ref, w3_ref, bg_ref, b1_ref, b3_ref,
              out_ref):
    cr = lax.dot_general(bg_ref[...], w1_ref[...], (((1,), (1,)), ((), ())),
                         preferred_element_type=jnp.float32) + b1_ref[...]
    cr = lax.dot_general(cr, w3_ref[...], (((1,), (1,)), ((), ())),
                         preferred_element_type=jnp.float32) + b3_ref[...]
    cn = c_ref[...]
    deg = (cn[0] + cn[1])[:, None] + 1.0
    a = a_ref[...]
    agg = a[0] + a[1] + h2_ref[...]
    out_ref[...] = agg * lax.rsqrt(deg) + cr


_epi_call = pl.pallas_call(
    _epi_body,
    out_shape=jax.ShapeDtypeStruct((N, DO), jnp.float32),
)


def kernel(x, edge_index, batch, W_gcn, b_gcn, W1, b1, W3, b3):
    ei = edge_index.reshape(2, NBT, EB)
    ones1 = jnp.ones((EB,), jnp.float32)
    zeros1 = jnp.zeros((N,), jnp.float32)
    zeros64 = jnp.zeros((N, DO), jnp.float32)

    cnts = _cnt_kernel(ei, ones1, zeros1)
    h2 = _mm_call(x, W_gcn, W1, W3, cnts)
    aggs = _scat_kernel(ei, h2, zeros64)
    return _epi_call(aggs, h2, cnts, W1, W3,
                     b_gcn.reshape(1, DI), b1.reshape(1, DO),
                     b3.reshape(1, DO))


# speedup vs baseline: 56.6886x; 1.0047x over previous
import functools

import jax
import jax.numpy as jnp
from jax import lax
from jax.experimental import pallas as pl
from jax.experimental.pallas import tpu as pltpu
from jax.experimental.pallas import tpu_sc as plsc

N = 10000
DI = 128
DO = 64
E = 320000
NC = 2
NS = 16
NW = NC * NS
EB = 128
NBT = E // EB
NBMAX = NBT // NW + 1
RPT = N // NS

_MESH = functools.partial(
    plsc.VectorSubcoreMesh,
    core_axis_name="c", subcore_axis_name="s", num_cores=NC, num_subcores=NS,
)
_SC_PARAMS = pltpu.CompilerParams(use_tc_tiling_on_sc=False)


def _tile_ids():
    c = lax.axis_index("c")
    s = lax.axis_index("s")
    wid = s * NC + c
    b0 = (wid * NBT) // NW
    nb = ((wid + 1) * NBT) // NW - b0
    return c, s, wid, b0, nb


@functools.partial(
    pl.kernel,
    out_type=jax.ShapeDtypeStruct((NC, N), jnp.float32),
    mesh=_MESH(),
    compiler_params=_SC_PARAMS,
    scratch_types=[
        pltpu.VMEM((NBMAX, EB), jnp.int32),
        pltpu.VMEM((EB,), jnp.float32),
        pltpu.VMEM_SHARED((N,), jnp.float32),
    ],
)
def _cnt_kernel(ei_hbm, ones_hbm, zeros_hbm, out_hbm, dst_v, ones_v, cnt_sh):
    c, s, wid, b0, nb = _tile_ids()
    r0 = lax.div(s * RPT, 8) * 8
    pltpu.sync_copy(zeros_hbm.at[pl.ds(r0, RPT + 7)],
                    cnt_sh.at[pl.ds(r0, RPT + 7)])
    pltpu.sync_copy(ones_hbm, ones_v)
    pltpu.sync_copy(ei_hbm.at[1, pl.ds(b0, NBMAX)], dst_v)
    plsc.subcore_barrier()

    def body(b, carry):
        pltpu.sync_copy(ones_v, cnt_sh.at[dst_v.at[b]], add=True)
        return carry

    lax.fori_loop(0, nb, body, 0)
    plsc.subcore_barrier()
    pltpu.sync_copy(cnt_sh.at[pl.ds(r0, RPT + 7)],
                    out_hbm.at[c, pl.ds(r0, RPT + 7)])


@functools.partial(
    pl.kernel,
    out_type=jax.ShapeDtypeStruct((NC, N, DO), jnp.float32),
    mesh=_MESH(),
    compiler_params=_SC_PARAMS,
    scratch_types=[
        pltpu.VMEM((NBMAX, EB), jnp.int32),
        pltpu.VMEM((NBMAX, EB), jnp.int32),
        pltpu.VMEM((3, EB, DO), jnp.float32),
        pltpu.VMEM_SHARED((N, DO), jnp.float32),
        pltpu.VMEM_SHARED((N, DO), jnp.float32),
        pltpu.SemaphoreType.DMA((3,)),
        pltpu.SemaphoreType.DMA((3,)),
    ],
)
def _scat_kernel(ei_hbm, h2_hbm, zeros_hbm, out_hbm,
                 src_v, dst_v, rows_v, agg_sh, h2_sh, semg, sems):
    c, s, wid, b0, nb = _tile_ids()
    base = s * RPT
    pltpu.sync_copy(zeros_hbm.at[pl.ds(base, RPT)], agg_sh.at[pl.ds(base, RPT)])
    pltpu.sync_copy(h2_hbm.at[pl.ds(base, RPT)], h2_sh.at[pl.ds(base, RPT)])
    pltpu.sync_copy(ei_hbm.at[0, pl.ds(b0, NBMAX)], src_v)
    pltpu.sync_copy(ei_hbm.at[1, pl.ds(b0, NBMAX)], dst_v)
    plsc.subcore_barrier()

    for j in range(3):
        pltpu.async_copy(h2_sh.at[src_v.at[j]], rows_v.at[j], semg.at[j])

    def body(b, carry):
        j = lax.rem(b, 3)
        pltpu.make_async_copy(h2_hbm.at[src_v.at[0]], rows_v.at[j],
                              semg.at[j]).wait()
        pltpu.async_copy(rows_v.at[j], agg_sh.at[dst_v.at[b]], sems.at[j],
                         add=True)
        j2 = lax.rem(b + 2, 3)

        @pl.when(b >= 1)
        def _():
            pltpu.make_async_copy(rows_v.at[j2], agg_sh.at[dst_v.at[0]],
                                  sems.at[j2]).wait()
            pltpu.async_copy(h2_sh.at[src_v.at[lax.rem(b + 2, nb)]],
                             rows_v.at[j2], semg.at[j2])

        return carry

    lax.fori_loop(0, nb, body, 0)
    js = lax.rem(nb + 2, 3)
    pltpu.make_async_copy(rows_v.at[js], agg_sh.at[dst_v.at[0]],
                          sems.at[js]).wait()
    for d in range(2):
        jg = lax.rem(nb + d, 3)
        pltpu.make_async_copy(h2_hbm.at[src_v.at[0]], rows_v.at[jg],
                              semg.at[jg]).wait()
    plsc.subcore_barrier()
    pltpu.sync_copy(agg_sh.at[pl.ds(base, RPT)], out_hbm.at[c, pl.ds(base, RPT)])


def _mm_body(x_ref, wg_ref, w1_ref, w3_ref, h_ref):
    h = lax.dot_general(x_ref[...], wg_ref[...], (((1,), (1,)), ((), ())),
                        preferred_element_type=jnp.float32)
    h = lax.dot_general(h, w1_ref[...], (((1,), (1,)), ((), ())),
                        preferred_element_type=jnp.float32)
    h_ref[...] = lax.dot_general(h, w3_ref[...], (((1,), (1,)), ((), ())),
                                 preferred_element_type=jnp.float32)


_mm_call = pl.pallas_call(
    _mm_body,
    out_shape=jax.ShapeDtypeStruct((N, DO), jnp.float32),
)


def _scale_body(h_ref, c_ref, h2_ref):
    cn = c_ref[...]
    cnt = (cn[0] + cn[1])[:, None]
    h2_ref[...] = h_ref[...] * lax.rsqrt(cnt + 1.0)


_scale_call = pl.pallas_call(
    _scale_body,
    out_shape=jax.ShapeDtypeStruct((N, DO), jnp.float32),
)


def _epi_body(a_ref, h2_ref, c_ref, w1_ref, w3_ref, bg_ref, b1_ref, b3_ref,
              out_ref):
    cr = lax.dot_general(bg_ref[...], w1_ref[...], (((1,), (1,)), ((), ())),
                         preferred_element_type=jnp.float32) + b1_ref[...]
    cr = lax.dot_general(cr, w3_ref[...], (((1,), (1,)), ((), ())),
                         preferred_element_type=jnp.float32) + b3_ref[...]
    cn = c_ref[...]
    deg = (cn[0] + cn[1])[:, None] + 1.0
    a = a_ref[...]
    agg = a[0] + a[1] + h2_ref[...]
    out_ref[...] = agg * lax.rsqrt(deg) + cr


_epi_call = pl.pallas_call(
    _epi_body,
    out_shape=jax.ShapeDtypeStruct((N, DO), jnp.float32),
)


def kernel(x, edge_index, batch, W_gcn, b_gcn, W1, b1, W3, b3):
    ei = edge_index.reshape(2, NBT, EB)
    ones1 = jnp.ones((EB,), jnp.float32)
    zeros1 = jnp.zeros((N,), jnp.float32)
    zeros64 = jnp.zeros((N, DO), jnp.float32)

    h_raw = _mm_call(x, W_gcn, W1, W3)
    cnts = _cnt_kernel(ei, ones1, zeros1)
    h2 = _scale_call(h_raw, cnts)
    aggs = _scat_kernel(ei, h2, zeros64)
    return _epi_call(aggs, h2, cnts, W1, W3,
                     b_gcn.reshape(1, DI), b1.reshape(1, DO),
                     b3.reshape(1, DO))


# speedup vs baseline: 57.9396x; 1.0221x over previous
import functools

import jax
import jax.numpy as jnp
from jax import lax
from jax.experimental import pallas as pl
from jax.experimental.pallas import tpu as pltpu
from jax.experimental.pallas import tpu_sc as plsc

N = 10000
DI = 128
DO = 64
E = 320000
NC = 2
NS = 16
NW = NC * NS
EB = 128
NBT = E // EB
NBMAX = NBT // NW + 1
RPT = N // NS

_MESH = functools.partial(
    plsc.VectorSubcoreMesh,
    core_axis_name="c", subcore_axis_name="s", num_cores=NC, num_subcores=NS,
)
_SC_PARAMS = pltpu.CompilerParams(use_tc_tiling_on_sc=False)


def _tile_ids():
    c = lax.axis_index("c")
    s = lax.axis_index("s")
    wid = s * NC + c
    b0 = (wid * NBT) // NW
    nb = ((wid + 1) * NBT) // NW - b0
    return c, s, wid, b0, nb


@functools.partial(
    pl.kernel,
    out_type=jax.ShapeDtypeStruct((NC, N), jnp.float32),
    mesh=_MESH(),
    compiler_params=_SC_PARAMS,
    scratch_types=[
        pltpu.VMEM((NBMAX, EB), jnp.int32),
        pltpu.VMEM((EB,), jnp.float32),
        pltpu.VMEM_SHARED((N,), jnp.float32),
        pltpu.SemaphoreType.DMA((2,)),
    ],
)
def _cnt_kernel(ei_hbm, ones_hbm, zeros_hbm, out_hbm, dst_v, ones_v, cnt_sh,
                sems):
    c, s, wid, b0, nb = _tile_ids()
    r0 = lax.div(s * RPT, 8) * 8
    pltpu.sync_copy(zeros_hbm.at[pl.ds(r0, RPT + 7)],
                    cnt_sh.at[pl.ds(r0, RPT + 7)])
    pltpu.sync_copy(ones_hbm, ones_v)
    pltpu.sync_copy(ei_hbm.at[1, pl.ds(b0, NBMAX)], dst_v)
    plsc.subcore_barrier()

    def body(b, carry):
        j = lax.rem(b, 2)

        @pl.when(b >= 2)
        def _():
            pltpu.make_async_copy(ones_v, cnt_sh.at[dst_v.at[0]],
                                  sems.at[j]).wait()

        pltpu.async_copy(ones_v, cnt_sh.at[dst_v.at[b]], sems.at[j], add=True)
        return carry

    lax.fori_loop(0, nb, body, 0)
    for d in range(2):
        jd = lax.rem(nb + d, 2)
        pltpu.make_async_copy(ones_v, cnt_sh.at[dst_v.at[0]],
                              sems.at[jd]).wait()
    plsc.subcore_barrier()
    pltpu.sync_copy(cnt_sh.at[pl.ds(r0, RPT + 7)],
                    out_hbm.at[c, pl.ds(r0, RPT + 7)])


@functools.partial(
    pl.kernel,
    out_type=jax.ShapeDtypeStruct((NC, N, DO), jnp.float32),
    mesh=_MESH(),
    compiler_params=_SC_PARAMS,
    scratch_types=[
        pltpu.VMEM((NBMAX, EB), jnp.int32),
        pltpu.VMEM((NBMAX, EB), jnp.int32),
        pltpu.VMEM((3, EB, DO), jnp.float32),
        pltpu.VMEM_SHARED((N, DO), jnp.float32),
        pltpu.VMEM_SHARED((N, DO), jnp.float32),
        pltpu.SemaphoreType.DMA((3,)),
        pltpu.SemaphoreType.DMA((3,)),
    ],
)
def _scat_kernel(ei_hbm, h2_hbm, zeros_hbm, out_hbm,
                 src_v, dst_v, rows_v, agg_sh, h2_sh, semg, sems):
    c, s, wid, b0, nb = _tile_ids()
    base = s * RPT
    pltpu.sync_copy(zeros_hbm.at[pl.ds(base, RPT)], agg_sh.at[pl.ds(base, RPT)])
    pltpu.sync_copy(h2_hbm.at[pl.ds(base, RPT)], h2_sh.at[pl.ds(base, RPT)])
    pltpu.sync_copy(ei_hbm.at[0, pl.ds(b0, NBMAX)], src_v)
    pltpu.sync_copy(ei_hbm.at[1, pl.ds(b0, NBMAX)], dst_v)
    plsc.subcore_barrier()

    for j in range(3):
        pltpu.async_copy(h2_sh.at[src_v.at[j]], rows_v.at[j], semg.at[j])

    def body(b, carry):
        j = lax.rem(b, 3)
        pltpu.make_async_copy(h2_hbm.at[src_v.at[0]], rows_v.at[j],
                              semg.at[j]).wait()
        pltpu.async_copy(rows_v.at[j], agg_sh.at[dst_v.at[b]], sems.at[j],
                         add=True)
        j2 = lax.rem(b + 2, 3)

        @pl.when(b >= 1)
        def _():
            pltpu.make_async_copy(rows_v.at[j2], agg_sh.at[dst_v.at[0]],
                                  sems.at[j2]).wait()
            pltpu.async_copy(h2_sh.at[src_v.at[lax.rem(b + 2, nb)]],
                             rows_v.at[j2], semg.at[j2])

        return carry

    lax.fori_loop(0, nb, body, 0)
    js = lax.rem(nb + 2, 3)
    pltpu.make_async_copy(rows_v.at[js], agg_sh.at[dst_v.at[0]],
                          sems.at[js]).wait()
    for d in range(2):
        jg = lax.rem(nb + d, 3)
        pltpu.make_async_copy(h2_hbm.at[src_v.at[0]], rows_v.at[jg],
                              semg.at[jg]).wait()
    plsc.subcore_barrier()
    pltpu.sync_copy(agg_sh.at[pl.ds(base, RPT)], out_hbm.at[c, pl.ds(base, RPT)])


def _mm_body(x_ref, wg_ref, w1_ref, w3_ref, h_ref):
    h = lax.dot_general(x_ref[...], wg_ref[...], (((1,), (1,)), ((), ())),
                        preferred_element_type=jnp.float32)
    h = lax.dot_general(h, w1_ref[...], (((1,), (1,)), ((), ())),
                        preferred_element_type=jnp.float32)
    h_ref[...] = lax.dot_general(h, w3_ref[...], (((1,), (1,)), ((), ())),
                                 preferred_element_type=jnp.float32)


_mm_call = pl.pallas_call(
    _mm_body,
    out_shape=jax.ShapeDtypeStruct((N, DO), jnp.float32),
)


def _scale_body(h_ref, c_ref, h2_ref):
    cn = c_ref[...]
    cnt = (cn[0] + cn[1])[:, None]
    h2_ref[...] = h_ref[...] * lax.rsqrt(cnt + 1.0)


_scale_call = pl.pallas_call(
    _scale_body,
    out_shape=jax.ShapeDtypeStruct((N, DO), jnp.float32),
)


def _epi_body(a_ref, h2_ref, c_ref, w1_ref, w3_ref, bg_ref, b1_ref, b3_ref,
              out_ref):
    cr = lax.dot_general(bg_ref[...], w1_ref[...], (((1,), (1,)), ((), ())),
                         preferred_element_type=jnp.float32) + b1_ref[...]
    cr = lax.dot_general(cr, w3_ref[...], (((1,), (1,)), ((), ())),
                         preferred_element_type=jnp.float32) + b3_ref[...]
    cn = c_ref[...]
    deg = (cn[0] + cn[1])[:, None] + 1.0
    a = a_ref[...]
    agg = a[0] + a[1] + h2_ref[...]
    out_ref[...] = agg * lax.rsqrt(deg) + cr


_epi_call = pl.pallas_call(
    _epi_body,
    out_shape=jax.ShapeDtypeStruct((N, DO), jnp.float32),
)


def kernel(x, edge_index, batch, W_gcn, b_gcn, W1, b1, W3, b3):
    ei = edge_index.reshape(2, NBT, EB)
    ones1 = jnp.ones((EB,), jnp.float32)
    zeros1 = jnp.zeros((N,), jnp.float32)
    zeros64 = jnp.zeros((N, DO), jnp.float32)

    h_raw = _mm_call(x, W_gcn, W1, W3)
    cnts = _cnt_kernel(ei, ones1, zeros1)
    h2 = _scale_call(h_raw, cnts)
    aggs = _scat_kernel(ei, h2, zeros64)
    return _epi_call(aggs, h2, cnts, W1, W3,
                     b_gcn.reshape(1, DI), b1.reshape(1, DO),
                     b3.reshape(1, DO))
